# Initial kernel scaffold; baseline (speedup 1.0000x reference)
import jax
import jax.numpy as jnp
from jax.experimental import pallas as pl


def kernel(x, z, training, emb, Wf, Uf, bf, Wb, Ub, bb, Wc, bc, Wd, bd, gamma, beta, Wl, Ul, bl):
    raise NotImplementedError("write your pallas kernel here")


# speedup vs baseline: 3.0691x; 3.0691x over previous
import jax
import jax.numpy as jnp
from jax.experimental import pallas as pl
from jax.experimental.pallas import tpu as pltpu

_B, _T, _DX = 64, 128, 192
_VOCAB, _EMB, _TXT = 184, 8, 5
_BI, _LU = 32, 10
_F, _H = 128, 1024


def _body(zi_ref, xt_ref, emb_ref, wf_ref, uf_ref, bf_ref, wb_ref, ub_ref,
          bb_ref, wcx0_ref, wcx1_ref, wcz0_ref, wcz1_ref, bc_ref, wd_ref,
          bd_ref, gam_ref, bet_ref, wl_ref, ul_ref, bl_ref, out_ref, gx_ref):
    f32 = jnp.float32

    def dot(a, b):
        return jax.lax.dot(a, b, preferred_element_type=f32)

    zi = zi_ref[...]
    oh = (zi == jax.lax.broadcasted_iota(jnp.int32, (_TXT * _B, _VOCAB), 1))
    ze = dot(oh.astype(f32), emb_ref[...])

    def lstm5(W, U, b, order):
        h = jnp.zeros((_B, _BI), f32)
        c = jnp.zeros((_B, _BI), f32)
        for t in order:
            xt = ze[t * _B:(t + 1) * _B, :]
            g = dot(xt, W) + dot(h, U) + b
            i = jax.nn.sigmoid(g[:, :_BI])
            f = jax.nn.sigmoid(g[:, _BI:2 * _BI])
            gc = jnp.tanh(g[:, 2 * _BI:3 * _BI])
            o = jax.nn.sigmoid(g[:, 3 * _BI:])
            c = f * c + i * gc
            h = o * jnp.tanh(c)
        return h

    hf = lstm5(wf_ref[...], uf_ref[...], bf_ref[...], range(_TXT))
    hb = lstm5(wb_ref[...], ub_ref[...], bb_ref[...], range(_TXT - 1, -1, -1))
    zcat = jnp.concatenate([hf, hb], axis=1)

    zrep = jnp.concatenate([zcat, zcat], axis=0)
    zsh = jnp.concatenate([zrep[1:], jnp.zeros((1, 2 * _BI), f32)], axis=0)
    zconv = dot(zrep, wcz0_ref[...]) + dot(zsh, wcz1_ref[...])

    X = xt_ref[...]
    a0 = dot(X, wcx0_ref[...])
    a1 = dot(X, wcx1_ref[...])
    a1s = jnp.concatenate([a1[_B:], jnp.zeros((_B, _F), f32)], axis=0)
    y = a0 + a1s + bc_ref[...]
    y = y.reshape(_T, _B, _F) + zconv[:, None, :]
    y = jnp.maximum(y, 0.0).reshape(_T * _B, _F)

    Wd = wd_ref[...]
    Wl = wl_ref[...]
    bd = bd_ref[...]
    gam = gam_ref[...]
    bet = bet_ref[...]
    bl = bl_ref[...]
    chunk = 2048
    for s in range(0, _T * _B, chunk):
        hh = dot(y[s:s + chunk], Wd) + bd
        mu = jnp.mean(hh, axis=1, keepdims=True)
        dz = hh - mu
        var = jnp.mean(dz * dz, axis=1, keepdims=True)
        hn = dz * jax.lax.rsqrt(var + 1e-6) * gam + bet
        gx_ref[s:s + chunk, :] = dot(hn, Wl) + bl

    Ul = ul_ref[...]

    def step(t, hc):
        h, c = hc
        g = gx_ref[pl.ds(t * _B, _B), :] + dot(h, Ul)
        i = jax.nn.sigmoid(g[:, :_LU])
        f = jax.nn.sigmoid(g[:, _LU:2 * _LU])
        gc = jnp.tanh(g[:, 2 * _LU:3 * _LU])
        o = jax.nn.sigmoid(g[:, 3 * _LU:])
        c = f * c + i * gc
        h = o * jnp.tanh(c)
        return (h, c)

    h0 = jnp.zeros((_B, _LU), f32)
    h, _ = jax.lax.fori_loop(0, _T, step, (h0, h0))
    out_ref[...] = h


def kernel(x, z, training, emb, Wf, Uf, bf, Wb, Ub, bb, Wc, bc, Wd, bd,
           gamma, beta, Wl, Ul, bl):
    del training
    xt = jnp.transpose(x, (1, 0, 2)).reshape(_T * _B, _DX)
    zi = jnp.transpose(z, (1, 0)).reshape(_TXT * _B, 1).astype(jnp.int32)
    args = (
        zi, xt, emb,
        Wf, Uf, bf.reshape(1, -1),
        Wb, Ub, bb.reshape(1, -1),
        Wc[0, :_DX, :], Wc[1, :_DX, :], Wc[0, _DX:, :], Wc[1, _DX:, :],
        bc.reshape(1, -1),
        Wd, bd.reshape(1, -1), gamma.reshape(1, -1), beta.reshape(1, -1),
        Wl, Ul, bl.reshape(1, -1),
    )
    return pl.pallas_call(
        _body,
        out_shape=jax.ShapeDtypeStruct((_B, _LU), jnp.float32),
        scratch_shapes=[pltpu.VMEM((_T * _B, 4 * _LU), jnp.float32)],
    )(*args)


# speedup vs baseline: 3.2033x; 1.0437x over previous
import jax
import jax.numpy as jnp
from jax.experimental import pallas as pl
from jax.experimental.pallas import tpu as pltpu

_B, _T, _DX = 64, 128, 192
_VOCAB, _EMB, _TXT = 184, 8, 5
_BI, _LU = 32, 10
_F, _H = 128, 1024


def _body(zi_ref, xt_ref, emb_ref, wf_ref, uf_ref, bf_ref, wb_ref, ub_ref,
          bb_ref, wcx0_ref, wcx1_ref, wcz0_ref, wcz1_ref, bc_ref, wd_ref,
          bd_ref, gam_ref, gamc_ref, bet_ref, wl_ref, ul_ref, bl_ref,
          out_ref, gx_ref):
    f32 = jnp.float32
    bf16 = jnp.bfloat16

    def dot(a, b):
        return jax.lax.dot(a, b, preferred_element_type=f32)

    zi = zi_ref[...]
    oh = (zi == jax.lax.broadcasted_iota(jnp.int32, (_TXT * _B, _VOCAB), 1))
    ze = dot(oh.astype(f32), emb_ref[...])

    def lstm5(W, U, b, order):
        h = jnp.zeros((_B, _BI), f32)
        c = jnp.zeros((_B, _BI), f32)
        for t in order:
            xt = ze[t * _B:(t + 1) * _B, :]
            g = dot(xt, W) + dot(h, U) + b
            i = jax.nn.sigmoid(g[:, :_BI])
            f = jax.nn.sigmoid(g[:, _BI:2 * _BI])
            gc = jnp.tanh(g[:, 2 * _BI:3 * _BI])
            o = jax.nn.sigmoid(g[:, 3 * _BI:])
            c = f * c + i * gc
            h = o * jnp.tanh(c)
        return h

    hf = lstm5(wf_ref[...], uf_ref[...], bf_ref[...], range(_TXT))
    hb = lstm5(wb_ref[...], ub_ref[...], bb_ref[...], range(_TXT - 1, -1, -1))
    zcat = jnp.concatenate([hf, hb], axis=1)

    zrep = jnp.concatenate([zcat, zcat], axis=0)
    zsh = jnp.concatenate([zrep[1:], jnp.zeros((1, 2 * _BI), f32)], axis=0)
    zconv = (dot(zrep, wcz0_ref[...]) + dot(zsh, wcz1_ref[...])
             + bc_ref[...])

    X = xt_ref[...].astype(bf16)
    a0 = dot(X, wcx0_ref[...].astype(bf16))
    a1 = dot(X, wcx1_ref[...].astype(bf16))
    a1s = jnp.concatenate([a1[_B:], jnp.zeros((_B, _F), f32)], axis=0)
    y = a0 + a1s
    y = y.reshape(_T, _B, _F) + zconv[:, None, :]
    y = jnp.maximum(y, 0.0).reshape(_T * _B, _F).astype(bf16)

    Wd = wd_ref[...].astype(bf16)
    Wl = wl_ref[...]
    bd = bd_ref[...]
    gam = gam_ref[...]
    bet = bet_ref[...]
    bl = bl_ref[...]
    Wlg = (gamc_ref[...] * Wl).astype(bf16)
    gwl = dot(gam, Wl)
    cst = dot(bet, Wl) + bl
    inv_h = 1.0 / _H
    chunk = 2048
    for s in range(0, _T * _B, chunk):
        hh = dot(y[s:s + chunk], Wd) + bd
        mu = jnp.sum(hh, axis=1, keepdims=True) * inv_h
        msq = jnp.sum(hh * hh, axis=1, keepdims=True) * inv_h
        rsig = jax.lax.rsqrt(msq - mu * mu + 1e-6)
        gx_ref[s:s + chunk, :] = (rsig * dot(hh.astype(bf16), Wlg)
                                  - (rsig * mu) * gwl + cst)

    Ul = ul_ref[...]

    def step(t, hc):
        h, c = hc
        g = gx_ref[pl.ds(t * _B, _B), :] + dot(h, Ul)
        i = jax.nn.sigmoid(g[:, :_LU])
        f = jax.nn.sigmoid(g[:, _LU:2 * _LU])
        gc = jnp.tanh(g[:, 2 * _LU:3 * _LU])
        o = jax.nn.sigmoid(g[:, 3 * _LU:])
        c = f * c + i * gc
        h = o * jnp.tanh(c)
        return (h, c)

    h0 = jnp.zeros((_B, _LU), f32)
    h, _ = jax.lax.fori_loop(0, _T, step, (h0, h0))
    out_ref[...] = h


def kernel(x, z, training, emb, Wf, Uf, bf, Wb, Ub, bb, Wc, bc, Wd, bd,
           gamma, beta, Wl, Ul, bl):
    del training
    xt = jnp.transpose(x, (1, 0, 2)).reshape(_T * _B, _DX)
    zi = jnp.transpose(z, (1, 0)).reshape(_TXT * _B, 1).astype(jnp.int32)
    args = (
        zi, xt, emb,
        Wf, Uf, bf.reshape(1, -1),
        Wb, Ub, bb.reshape(1, -1),
        Wc[0, :_DX, :], Wc[1, :_DX, :], Wc[0, _DX:, :], Wc[1, _DX:, :],
        bc.reshape(1, -1),
        Wd, bd.reshape(1, -1), gamma.reshape(1, -1), gamma.reshape(-1, 1),
        beta.reshape(1, -1),
        Wl, Ul, bl.reshape(1, -1),
    )
    return pl.pallas_call(
        _body,
        out_shape=jax.ShapeDtypeStruct((_B, _LU), jnp.float32),
        scratch_shapes=[pltpu.VMEM((_T * _B, 4 * _LU), jnp.float32)],
    )(*args)


# speedup vs baseline: 4.7533x; 1.4839x over previous
import jax
import jax.numpy as jnp
from jax.experimental import pallas as pl
from jax.experimental.pallas import tpu as pltpu

_B, _T, _DX = 64, 128, 192
_VOCAB, _EMB, _TXT = 184, 8, 5
_BI, _LU = 32, 10
_F, _H = 128, 1024


def _body(zi_ref, xt_ref, emb_ref, wf_ref, uf_ref, bf_ref, wb_ref, ub_ref,
          bb_ref, wcx0_ref, wcx1_ref, wcz0_ref, wcz1_ref, bc_ref, wd_ref,
          bd_ref, gam_ref, gamc_ref, bet_ref, wl_ref, ulp_ref, bl_ref,
          out_ref, gx_ref):
    f32 = jnp.float32
    bf16 = jnp.bfloat16

    def dot(a, b):
        return jax.lax.dot(a, b, preferred_element_type=f32)

    zi = zi_ref[...]
    oh = (zi == jax.lax.broadcasted_iota(jnp.int32, (_TXT * _B, _VOCAB), 1))
    ze = dot(oh.astype(f32), emb_ref[...])

    def lstm5(W, U, b, order):
        h = jnp.zeros((_B, _BI), f32)
        c = jnp.zeros((_B, _BI), f32)
        for t in order:
            xt = ze[t * _B:(t + 1) * _B, :]
            g = dot(xt, W) + dot(h, U) + b
            i = jax.nn.sigmoid(g[:, :_BI])
            f = jax.nn.sigmoid(g[:, _BI:2 * _BI])
            gc = jnp.tanh(g[:, 2 * _BI:3 * _BI])
            o = jax.nn.sigmoid(g[:, 3 * _BI:])
            c = f * c + i * gc
            h = o * jnp.tanh(c)
        return h

    hf = lstm5(wf_ref[...], uf_ref[...], bf_ref[...], range(_TXT))
    hb = lstm5(wb_ref[...], ub_ref[...], bb_ref[...], range(_TXT - 1, -1, -1))
    zcat = jnp.concatenate([hf, hb], axis=1)

    zrep = jnp.concatenate([zcat, zcat], axis=0)
    zsh = jnp.concatenate([zrep[1:], jnp.zeros((1, 2 * _BI), f32)], axis=0)
    zconv = (dot(zrep, wcz0_ref[...]) + dot(zsh, wcz1_ref[...])
             + bc_ref[...])

    X = xt_ref[...].astype(bf16)
    a0 = dot(X, wcx0_ref[...].astype(bf16))
    a1 = dot(X, wcx1_ref[...].astype(bf16))
    a1s = jnp.concatenate([a1[_B:], jnp.zeros((_B, _F), f32)], axis=0)
    y = a0 + a1s
    y = y.reshape(_T, _B, _F) + zconv[:, None, :]
    y = jnp.maximum(y, 0.0).reshape(_T * _B, _F).astype(bf16)

    GP4 = 4 * 16
    Wd = wd_ref[...].astype(bf16)
    Wl = wl_ref[...]
    bd = bd_ref[...]
    gam = gam_ref[...]
    bet = bet_ref[...]
    bl = bl_ref[...]
    Wlg = (gamc_ref[...] * Wl).astype(bf16)
    gwl = dot(gam, Wl)
    cst = dot(bet, Wl) + bl
    inv_h = 1.0 / _H
    chunk = 2048
    tpc = chunk // _B
    for s in range(0, _T * _B, chunk):
        hh = dot(y[s:s + chunk], Wd) + bd
        mu = jnp.sum(hh, axis=1, keepdims=True) * inv_h
        msq = jnp.sum(hh * hh, axis=1, keepdims=True) * inv_h
        rsig = jax.lax.rsqrt(msq - mu * mu + 1e-6)
        gxc = rsig * dot(hh.astype(bf16), Wlg) - (rsig * mu) * gwl + cst
        gxt = jnp.transpose(gxc.reshape(tpc, _B, GP4), (0, 2, 1))
        gx_ref[s:s + chunk, :] = gxt.reshape(chunk, _B)

    Ulp = ulp_ref[...]

    def step(t, hc):
        h, c = hc
        g = gx_ref[pl.ds(t * GP4, GP4), :] + dot(Ulp, h)
        sif = jax.nn.sigmoid(g[0:32, :])
        gc = jnp.tanh(g[32:48, :])
        o = jax.nn.sigmoid(g[48:64, :])
        c = sif[16:32, :] * c + sif[0:16, :] * gc
        h = o * jnp.tanh(c)
        return (h, c)

    h0 = jnp.zeros((16, _B), f32)
    h, _ = jax.lax.fori_loop(0, _T, step, (h0, h0))
    out_ref[...] = jnp.transpose(h, (1, 0))[:, :_LU]


def kernel(x, z, training, emb, Wf, Uf, bf, Wb, Ub, bb, Wc, bc, Wd, bd,
           gamma, beta, Wl, Ul, bl):
    del training
    xt = jnp.transpose(x, (1, 0, 2)).reshape(_T * _B, _DX)
    zi = jnp.transpose(z, (1, 0)).reshape(_TXT * _B, 1).astype(jnp.int32)
    args = (
        zi, xt, emb,
        Wf, Uf, bf.reshape(1, -1),
        Wb, Ub, bb.reshape(1, -1),
        Wc[0, :_DX, :], Wc[1, :_DX, :], Wc[0, _DX:, :], Wc[1, _DX:, :],
        bc.reshape(1, -1),
        Wd, bd.reshape(1, -1), gamma.reshape(1, -1), gamma.reshape(-1, 1),
        beta.reshape(1, -1),
        jnp.pad(Wl.reshape(_H, 4, _LU), ((0, 0), (0, 0), (0, 6))).reshape(_H, 64),
        jnp.pad(Ul.T.reshape(4, _LU, _LU), ((0, 0), (0, 6), (0, 6))).reshape(64, 16),
        jnp.pad(bl.reshape(4, _LU), ((0, 0), (0, 6))).reshape(1, 64),
    )
    return pl.pallas_call(
        _body,
        out_shape=jax.ShapeDtypeStruct((_B, _LU), jnp.float32),
        scratch_shapes=[pltpu.VMEM((_T * 64, _B), jnp.float32)],
    )(*args)


# speedup vs baseline: 5.1023x; 1.0734x over previous
import jax
import jax.numpy as jnp
from jax.experimental import pallas as pl
from jax.experimental.pallas import tpu as pltpu

_B, _T, _DX = 64, 128, 192
_VOCAB, _EMB, _TXT = 184, 8, 5
_BI, _LU = 32, 10
_F, _H = 128, 1024


def _body(zi_ref, xb_ref, emb_ref, wf_ref, uf_ref, bf_ref, wb_ref, ub_ref,
          bb_ref, wcx0_ref, wcx1_ref, wcz0_ref, wcz1_ref, bc_ref, wd_ref,
          bd_ref, bdc_ref, gam_ref, gamc_ref, bet_ref, wl_ref, ulp_ref,
          bl_ref, out_ref, gx_ref):
    f32 = jnp.float32
    bf16 = jnp.bfloat16

    def dot(a, b):
        return jax.lax.dot(a, b, preferred_element_type=f32)

    zi = zi_ref[...]
    oh = (zi == jax.lax.broadcasted_iota(jnp.int32, (_TXT * _B, _VOCAB), 1))
    ze = dot(oh.astype(f32), emb_ref[...])

    def lstm5(W, U, b, order):
        h = jnp.zeros((_B, _BI), f32)
        c = jnp.zeros((_B, _BI), f32)
        for t in order:
            xt = ze[t * _B:(t + 1) * _B, :]
            g = dot(xt, W) + dot(h, U) + b
            i = jax.nn.sigmoid(g[:, :_BI])
            f = jax.nn.sigmoid(g[:, _BI:2 * _BI])
            gc = jnp.tanh(g[:, 2 * _BI:3 * _BI])
            o = jax.nn.sigmoid(g[:, 3 * _BI:])
            c = f * c + i * gc
            h = o * jnp.tanh(c)
        return h

    hf = lstm5(wf_ref[...], uf_ref[...], bf_ref[...], range(_TXT))
    hb = lstm5(wb_ref[...], ub_ref[...], bb_ref[...], range(_TXT - 1, -1, -1))
    zcat = jnp.concatenate([hf, hb], axis=1)

    zrep = jnp.concatenate([zcat, zcat], axis=0)
    zsh = jnp.concatenate([zrep[1:], jnp.zeros((1, 2 * _BI), f32)], axis=0)
    zconv = (dot(zrep, wcz0_ref[...]) + dot(zsh, wcz1_ref[...])
             + bc_ref[...])

    X = xb_ref[...].astype(bf16)
    a0 = dot(X, wcx0_ref[...].astype(bf16))
    a1 = dot(X, wcx1_ref[...].astype(bf16))
    a1s = jnp.concatenate([a1[1:], jnp.zeros((1, _F), f32)], axis=0)
    rows = jax.lax.broadcasted_iota(jnp.int32, (_B * _T, 1), 0)
    a1s = jnp.where(rows % _T == _T - 1, 0.0, a1s)
    y3 = (a0 + a1s).reshape(_B, _T, _F) + zconv[None, :, :]
    y3 = jnp.maximum(y3, 0.0).astype(bf16)
    y = jnp.transpose(y3, (1, 0, 2)).reshape(_T * _B, _F)

    GP4 = 4 * 16
    Wd = wd_ref[...]
    Wl = wl_ref[...]
    bd = bd_ref[...]
    bdc = bdc_ref[...]
    gam = gam_ref[...]
    bet = bet_ref[...]
    bl = bl_ref[...]
    Wlg = gamc_ref[...] * Wl
    WdWlg = dot(Wd, Wlg)
    G = jax.lax.dot_general(Wd, Wd, (((1,), (1,)), ((), ())),
                            preferred_element_type=f32)
    wdsum = jnp.sum(Wd, axis=1).reshape(1, _F)
    wdbd = dot(Wd, bdc).reshape(1, _F)
    sum_bd = jnp.sum(bd)
    sum_bd2 = jnp.sum(bd * bd)
    bdWlg = dot(bd, Wlg)
    gwl = dot(gam, Wl)
    cst = dot(bet, Wl) + bl
    inv_h = 1.0 / _H

    Rm = jnp.concatenate([G, WdWlg], axis=1).astype(bf16)
    R = dot(y, Rm)
    P = R[:, :_F]
    q = R[:, _F:_F + 64]
    yf = y.astype(f32)
    mu = (jnp.sum(yf * wdsum, axis=1, keepdims=True) + sum_bd) * inv_h
    msq = (jnp.sum(P * yf, axis=1, keepdims=True)
           + 2.0 * jnp.sum(yf * wdbd, axis=1, keepdims=True)
           + sum_bd2) * inv_h
    rsig = jax.lax.rsqrt(msq - mu * mu + 1e-6)
    gx = rsig * (q + bdWlg) - (rsig * mu) * gwl + cst
    gx_ref[...] = jnp.transpose(gx.reshape(_T, _B, GP4),
                                (0, 2, 1)).reshape(_T * GP4, _B)

    Ulp = ulp_ref[...]

    def step(t, hc):
        h, c = hc
        g = gx_ref[pl.ds(t * GP4, GP4), :] + dot(Ulp, h)
        sif = jax.nn.sigmoid(g[0:32, :])
        gc = jnp.tanh(g[32:48, :])
        o = jax.nn.sigmoid(g[48:64, :])
        c = sif[16:32, :] * c + sif[0:16, :] * gc
        h = o * jnp.tanh(c)
        return (h, c)

    h0 = jnp.zeros((16, _B), f32)
    h, _ = jax.lax.fori_loop(0, _T, step, (h0, h0))
    out_ref[...] = jnp.transpose(h, (1, 0))[:, :_LU]


def kernel(x, z, training, emb, Wf, Uf, bf, Wb, Ub, bb, Wc, bc, Wd, bd,
           gamma, beta, Wl, Ul, bl):
    del training
    xb = x.reshape(_B * _T, _DX)
    zi = jnp.transpose(z, (1, 0)).reshape(_TXT * _B, 1).astype(jnp.int32)
    args = (
        zi, xb, emb,
        Wf, Uf, bf.reshape(1, -1),
        Wb, Ub, bb.reshape(1, -1),
        Wc[0, :_DX, :], Wc[1, :_DX, :], Wc[0, _DX:, :], Wc[1, _DX:, :],
        bc.reshape(1, -1),
        Wd, bd.reshape(1, -1), bd.reshape(-1, 1),
        gamma.reshape(1, -1), gamma.reshape(-1, 1), beta.reshape(1, -1),
        jnp.pad(Wl.reshape(_H, 4, _LU), ((0, 0), (0, 0), (0, 6))).reshape(_H, 64),
        jnp.pad(Ul.T.reshape(4, _LU, _LU), ((0, 0), (0, 6), (0, 6))).reshape(64, 16),
        jnp.pad(bl.reshape(4, _LU), ((0, 0), (0, 6))).reshape(1, 64),
    )
    return pl.pallas_call(
        _body,
        out_shape=jax.ShapeDtypeStruct((_B, _LU), jnp.float32),
        scratch_shapes=[pltpu.VMEM((_T * 64, _B), jnp.float32)],
    )(*args)
